# baseline (speedup 1.0000x reference)
import functools

import jax
import jax.numpy as jnp
from jax import lax
from jax.experimental import pallas as pl
from jax.experimental.pallas import tpu as pltpu
from jax.experimental.pallas import tpu_sc as plsc

LNUM, N, E, D, GH, LH, NSEG = 4, 10000, 320000, 128, 128, 256, 64
NC, NS = 2, 16
NW = NC * NS
EPW = E // NW
CH = 64
EPWP = 10240
NCHUNK = EPWP // CH
SSPLIT = 8
NCH_S = NCHUNK // SSPLIT
NP = N + 16
RPT = 624
ZR = 48
VLANES = 16


def _sc_conv_call(tableb, src4, dst4, ea4, w_row):
    mesh = plsc.VectorSubcoreMesh(core_axis_name="c", subcore_axis_name="s",
                                  num_cores=NC, num_subcores=NS)

    @functools.partial(
        pl.kernel,
        out_type=jax.ShapeDtypeStruct((NC, LNUM, N, D), jnp.float32),
        mesh=mesh,
        scratch_types=[
            pltpu.VMEM((NCH_S, CH), jnp.int32),
            pltpu.VMEM((NCH_S, CH), jnp.int32),
            pltpu.VMEM((NCH_S, CH), jnp.float32),
            pltpu.VMEM((CH, D), jnp.float32),
            pltpu.VMEM((CH, D), jnp.float32),
            pltpu.VMEM((CH, D), jnp.float32),
            pltpu.VMEM((D,), jnp.float32),
            pltpu.VMEM_SHARED((NP, D), jnp.float32),
            pltpu.SemaphoreType.DMA,
            pltpu.SemaphoreType.DMA,
            pltpu.SemaphoreType.DMA,
            pltpu.SemaphoreType.DMA,
            pltpu.SemaphoreType.DMA,
            pltpu.SemaphoreType.DMA,
        ],
    )
    def body(tab_hbm, src_hbm, dst_hbm, ea_hbm, w_hbm, out_hbm,
             src_v, dst_v, ea_v, b0, b1, b2, w_v, agg_sh,
             gsem0, gsem1, gsem2, ssem0, ssem1, ssem2):
        c = lax.axis_index("c")
        s = lax.axis_index("s")
        wid = c * NS + s
        ring = ((b0, gsem0, ssem0), (b1, gsem1, ssem1), (b2, gsem2, ssem2))

        pltpu.sync_copy(w_hbm.at[0], w_v)
        w_vecs = [w_v[pl.ds(16 * j, 16)] for j in range(D // VLANES)]
        zv = jnp.zeros((VLANES,), jnp.float32)

        def _zero_rows(r, carry):
            for j in range(D // VLANES):
                b0[r, pl.ds(16 * j, 16)] = zv
            return carry

        def _gather(k, buf, gs):
            pltpu.async_copy(tab_hbm.at[src_v.at[k]], buf, gs)

        def _gwait(k, buf, gs):
            pltpu.make_async_copy(tab_hbm.at[src_v.at[k]], buf, gs).wait()

        def _scat(k, buf, ss):
            pltpu.async_copy(buf, agg_sh.at[dst_v.at[k]], ss, add=True)

        def _swait(k, buf, ss):
            pltpu.make_async_copy(buf, agg_sh.at[dst_v.at[k]], ss).wait()

        def _layer(l, lcarry):
            lax.fori_loop(0, CH, _zero_rows, 0)
            for k in range(RPT // ZR):
                pltpu.sync_copy(
                    b0.at[pl.ds(0, ZR)],
                    agg_sh.at[pl.ds(s * RPT + k * ZR, ZR)])

            @pl.when(s == NS - 1)
            def _():
                pltpu.sync_copy(b0.at[pl.ds(0, N - NS * RPT)],
                                agg_sh.at[pl.ds(NS * RPT, N - NS * RPT)])
            plsc.subcore_barrier()

            def _compute(buf, g):
                def _edge16(i16, icarry):
                    ev = ea_v[g, pl.ds(i16 * VLANES, VLANES)]
                    ev = lax.min(lax.max(ev, 0.0), 1.0)
                    for k in range(VLANES):
                        e = ev[k]
                        ri = i16 * VLANES + k
                        for j in range(D // VLANES):
                            r = buf[ri, pl.ds(16 * j, 16)]
                            buf[ri, pl.ds(16 * j, 16)] = lax.max(
                                r + e * w_vecs[j], 0.0)
                    return icarry
                lax.fori_loop(0, CH // VLANES, _edge16, 0)

            def _sub(sb, scarry):
                pltpu.sync_copy(src_hbm.at[l, wid, sb], src_v)
                pltpu.sync_copy(dst_hbm.at[l, wid, sb], dst_v)
                pltpu.sync_copy(ea_hbm.at[l, wid, sb], ea_v)
                _gather(0, b0, gsem0)
                _gather(1, b1, gsem1)
                _gwait(0, b0, gsem0)
                _compute(b0, 0)
                _scat(0, b0, ssem0)
                _gather(2, b2, gsem2)
                _gwait(1, b1, gsem1)
                _compute(b1, 1)
                _scat(1, b1, ssem1)
                _swait(0, b0, ssem0)
                _gather(3, b0, gsem0)

                def _tri(p, carry):
                    for q in range(3):
                        k = 2 + 3 * p + q
                        buf, gs, ss = ring[(2 + q) % 3]
                        pbuf, pgs, pss = ring[(1 + q) % 3]
                        _gwait(k, buf, gs)
                        _compute(buf, k)
                        _scat(k, buf, ss)
                        _swait(k - 1, pbuf, pss)

                        @pl.when(k <= NCH_S - 3)
                        def _():
                            _gather(k + 2, pbuf, pgs)
                    return carry
                lax.fori_loop(0, (NCH_S - 2) // 3, _tri, 0)
                _swait(NCH_S - 1, b1, ssem1)
                return scarry
            lax.fori_loop(0, SSPLIT, _sub, 0)
            plsc.subcore_barrier()
            pltpu.sync_copy(agg_sh.at[pl.ds(s * RPT, RPT)],
                            out_hbm.at[c, l, pl.ds(s * RPT, RPT)])

            @pl.when(s == NS - 1)
            def _():
                pltpu.sync_copy(
                    agg_sh.at[pl.ds(NS * RPT, N - NS * RPT)],
                    out_hbm.at[c, l, pl.ds(NS * RPT, N - NS * RPT)])
            plsc.subcore_barrier()
            return lcarry
        lax.fori_loop(0, LNUM, _layer, 0)

    return body(tableb, src4, dst4, ea4, w_row)


NBM = 2000


def _prep_body(x_ref, b_ref, out_ref):
    out_ref[...] = x_ref[...] + b_ref[...][None]


def _prep_call(x, be1):
    return pl.pallas_call(
        _prep_body,
        grid=(LNUM, N // NBM),
        in_specs=[
            pl.BlockSpec((1, NBM, D), lambda l, i: (l, i, 0)),
            pl.BlockSpec((1, D), lambda l, i: (0, 0)),
        ],
        out_specs=pl.BlockSpec((1, NBM, D), lambda l, i: (l, i, 0)),
        out_shape=jax.ShapeDtypeStruct((LNUM, N, D), jnp.float32),
    )(x, be1)


def _mlp1_body(xb_ref, parts_ref, be1_ref, w1_ref, b1_ref, w2_ref, b2_ref,
               be2_ref, out_ref):
    h0 = (xb_ref[0] - be1_ref[...] + parts_ref[0, 0] + parts_ref[1, 0])
    a = jnp.maximum(
        jnp.dot(h0, w1_ref[...], preferred_element_type=jnp.float32)
        + b1_ref[...], 0.0)
    h1 = jnp.maximum(
        jnp.dot(a, w2_ref[...], preferred_element_type=jnp.float32)
        + b2_ref[...], 0.0)
    out_ref[0] = h1 + be2_ref[...]


def _mlp1_call(xb, parts, be1, w1t, b1, w2t, b2, be2):
    return pl.pallas_call(
        _mlp1_body,
        grid=(LNUM, N // NBM),
        in_specs=[
            pl.BlockSpec((1, NBM, D), lambda l, i: (l, i, 0)),
            pl.BlockSpec((2, 1, NBM, D), lambda l, i: (0, l, i, 0)),
            pl.BlockSpec((1, D), lambda l, i: (0, 0)),
            pl.BlockSpec((D, GH), lambda l, i: (0, 0)),
            pl.BlockSpec((1, GH), lambda l, i: (0, 0)),
            pl.BlockSpec((GH, GH), lambda l, i: (0, 0)),
            pl.BlockSpec((1, GH), lambda l, i: (0, 0)),
            pl.BlockSpec((1, GH), lambda l, i: (0, 0)),
        ],
        out_specs=pl.BlockSpec((1, NBM, GH), lambda l, i: (l, i, 0)),
        out_shape=jax.ShapeDtypeStruct((LNUM, N, GH), jnp.float32),
    )(xb, parts, be1, w1t, b1, w2t, b2, be2)


def _mlp2_body(h1b_ref, parts_ref, batch_ref, be2_ref, w1_ref, b1_ref,
               w2_ref, b2_ref, seg_ref, cnt_ref):
    l = pl.program_id(0)
    i = pl.program_id(1)
    h0 = (h1b_ref[0] - be2_ref[...] + parts_ref[0, 0] + parts_ref[1, 0])
    a = jnp.maximum(
        jnp.dot(h0, w1_ref[...], preferred_element_type=jnp.float32)
        + b1_ref[...], 0.0)
    h2 = (jnp.dot(a, w2_ref[...], preferred_element_type=jnp.float32)
          + b2_ref[...])
    b_blk = batch_ref[0, 0, :]
    io = lax.broadcasted_iota(jnp.int32, (NSEG, NBM), 0)
    oh = (b_blk[None, :] == io).astype(jnp.float32)
    sp = jnp.dot(oh, h2, preferred_element_type=jnp.float32)

    @pl.when(i == 0)
    def _():
        seg_ref[0] = sp

    @pl.when(i != 0)
    def _():
        seg_ref[0] = seg_ref[0] + sp

    cb = jnp.broadcast_to(jnp.sum(oh, axis=1, keepdims=True), (NSEG, GH))

    @pl.when((l == 0) & (i == 0))
    def _():
        cnt_ref[...] = cb

    @pl.when((l == 0) & (i != 0))
    def _():
        cnt_ref[...] = cnt_ref[...] + cb


def _mlp2_call(h1b, parts2, batch3, be2, w1t, b1, w2t, b2):
    return pl.pallas_call(
        _mlp2_body,
        grid=(LNUM, N // NBM),
        in_specs=[
            pl.BlockSpec((1, NBM, GH), lambda l, i: (l, i, 0)),
            pl.BlockSpec((2, 1, NBM, GH), lambda l, i: (0, l, i, 0)),
            pl.BlockSpec((1, 1, NBM), lambda l, i: (i, 0, 0)),
            pl.BlockSpec((1, GH), lambda l, i: (0, 0)),
            pl.BlockSpec((GH, GH), lambda l, i: (0, 0)),
            pl.BlockSpec((1, GH), lambda l, i: (0, 0)),
            pl.BlockSpec((GH, GH), lambda l, i: (0, 0)),
            pl.BlockSpec((1, GH), lambda l, i: (0, 0)),
        ],
        out_specs=[
            pl.BlockSpec((1, NSEG, GH), lambda l, i: (l, 0, 0)),
            pl.BlockSpec((NSEG, GH), lambda l, i: (0, 0)),
        ],
        out_shape=[
            jax.ShapeDtypeStruct((LNUM, NSEG, GH), jnp.float32),
            jax.ShapeDtypeStruct((NSEG, GH), jnp.float32),
        ],
    )(h1b, parts2, batch3, be2, w1t, b1, w2t, b2)


def _head_body(seg_ref, cnt_ref, mask_ref,
               wih1f_ref, whh1f_ref, bi1f_ref,
               wih1b_ref, whh1b_ref, bi1b_ref,
               wih2f_ref, whh2f_ref, bi2f_ref,
               wih2b_ref, whh2b_ref, bi2b_ref,
               fc1_ref, fb1_ref, fc2_ref, fb2_ref, w3_ref, b3_ref,
               out_ref):
    cnt = jnp.maximum(cnt_ref[...], 1.0)
    xs = [seg_ref[t] / cnt * mask_ref[t] for t in range(LNUM)]

    def lstm_dir(inp, wih_ref, whh_ref, bias_ref, reverse):
        h = jnp.zeros((NSEG, LH), jnp.float32)
        c = jnp.zeros((NSEG, LH), jnp.float32)
        ys = [None] * LNUM
        order = range(LNUM - 1, -1, -1) if reverse else range(LNUM)
        for t in order:
            g = (jnp.dot(inp[t], wih_ref[...],
                         preferred_element_type=jnp.float32)
                 + jnp.dot(h, whh_ref[...],
                           preferred_element_type=jnp.float32)
                 + bias_ref[...])
            ig = jax.nn.sigmoid(g[:, 0:LH])
            fg = jax.nn.sigmoid(g[:, LH:2 * LH])
            gg = jnp.tanh(g[:, 2 * LH:3 * LH])
            og = jax.nn.sigmoid(g[:, 3 * LH:4 * LH])
            c = fg * c + ig * gg
            h = og * jnp.tanh(c)
            ys[t] = h
        return ys, h

    ysf, _ = lstm_dir(xs, wih1f_ref, whh1f_ref, bi1f_ref, False)
    ysb, _ = lstm_dir(xs, wih1b_ref, whh1b_ref, bi1b_ref, True)
    xs2 = [jnp.concatenate([ysf[t], ysb[t]], axis=1) for t in range(LNUM)]
    _, hf2 = lstm_dir(xs2, wih2f_ref, whh2f_ref, bi2f_ref, False)
    _, hb2 = lstm_dir(xs2, wih2b_ref, whh2b_ref, bi2b_ref, True)
    final = jnp.concatenate([hf2, hb2], axis=1)
    z = jnp.maximum(
        jnp.dot(final, fc1_ref[...], preferred_element_type=jnp.float32)
        + fb1_ref[...], 0.0)
    z = jnp.maximum(
        jnp.dot(z, fc2_ref[...], preferred_element_type=jnp.float32)
        + fb2_ref[...], 0.0)
    res = jnp.sum(z * w3_ref[...], axis=1, keepdims=True)
    out_ref[...] = jnp.broadcast_to(res, (NSEG, GH)) + b3_ref[...]


def _head_call(seg, cnt, mask, lstm_w, fc1t, fb1, fc2t, fb2, w3, b3b):
    return pl.pallas_call(
        _head_body,
        out_shape=jax.ShapeDtypeStruct((NSEG, GH), jnp.float32),
    )(seg, cnt, mask, *lstm_w, fc1t, fb1, fc2t, fb2, w3, b3b)


def kernel(x, edge_index, edge_attr, batch, num_layers, params):
    f32 = jnp.float32
    pad = ((0, 0), (0, 0), (0, EPWP - EPW))
    src = jnp.pad(edge_index[:, 0, :].astype(jnp.int32).reshape(
        LNUM, NW, EPW), pad)
    dst = jnp.pad(edge_index[:, 1, :].astype(jnp.int32).reshape(
        LNUM, NW, EPW), pad, constant_values=N)
    ea = jnp.pad(edge_attr[:, :, 0].reshape(LNUM, NW, EPW), pad)
    offs = (jnp.arange(LNUM, dtype=jnp.int32) * N)[:, None, None]
    src4 = (src + offs).reshape(LNUM, NW, SSPLIT, NCH_S, CH)
    dst4 = dst.reshape(LNUM, NW, SSPLIT, NCH_S, CH)
    ea4 = ea.reshape(LNUM, NW, SSPLIT, NCH_S, CH)
    batch3 = batch.astype(jnp.int32).reshape(N // NBM, 1, NBM)

    we1, be1 = params['lin_edge1']
    we2, be2 = params['lin_edge2']
    we1r = we1.reshape(1, D)
    we2r = we2.reshape(1, GH)
    be1r = be1.reshape(1, D)
    be2r = be2.reshape(1, GH)
    (w11, b11), (w12, b12) = params['mlp1']
    (w21, b21), (w22, b22) = params['mlp2']
    w11t, w12t = w11.T, w12.T
    w21t, w22t = w21.T, w22.T
    b11r, b12r = b11.reshape(1, GH), b12.reshape(1, GH)
    b21r, b22r = b21.reshape(1, GH), b22.reshape(1, GH)

    lstm_w = []
    for (lp, inf) in ((params['lstm'][0][0], GH), (params['lstm'][0][1], GH),
                      (params['lstm'][1][0], 2 * LH),
                      (params['lstm'][1][1], 2 * LH)):
        Wih, Whh, bih, bhh = lp
        lstm_w += [Wih.T, Whh.T, (bih + bhh).reshape(1, 4 * LH)]
    fc1, fb1 = params['fc1']
    fc2, fb2 = params['fc2']
    fc3, fb3 = params['fc3']
    fc1t, fc2t = fc1.T, fc2.T
    fb1r, fb2r = fb1.reshape(1, 128), fb2.reshape(1, 64)
    w3r = fc3.reshape(1, 64)
    b3b = jnp.broadcast_to(fb3.reshape(1, 1), (1, GH))

    mask = (jnp.arange(LNUM) < num_layers).astype(f32)
    mask2 = jnp.broadcast_to(mask[:, None], (LNUM, GH))

    xb = _prep_call(x, be1r)
    parts1 = _sc_conv_call(xb.reshape(LNUM * N, D), src4, dst4, ea4, we1r)
    h1b = _mlp1_call(xb, parts1, be1r, w11t, b11r, w12t, b12r, be2r)
    parts2 = _sc_conv_call(h1b.reshape(LNUM * N, GH), src4, dst4, ea4, we2r)
    seg, cnt = _mlp2_call(h1b, parts2, batch3, be2r,
                          w21t, b21r, w22t, b22r)
    out = _head_call(seg, cnt, mask2, lstm_w, fc1t, fb1r, fc2t, fb2r,
                     w3r, b3b)
    return out[:, :1]


# speedup vs baseline: 2.6925x; 2.6925x over previous
import functools

import jax
import jax.numpy as jnp
from jax import lax
from jax.experimental import pallas as pl
from jax.experimental.pallas import tpu as pltpu
from jax.experimental.pallas import tpu_sc as plsc

LNUM, N, E, D, GH, LH, NSEG = 4, 10000, 320000, 128, 128, 256, 64
NC, NS = 2, 16
NW = NC * NS
EPW = E // NW
CH = 80
EPWP = EPW
NCHUNK = EPWP // CH
SSPLIT = 5
NCH_S = NCHUNK // SSPLIT
NP = N
RPT = 624
ZR = 48
VLANES = 16


def _sc_conv_call(tableb, src4, dst4, ea4, w_row):
    mesh = plsc.VectorSubcoreMesh(core_axis_name="c", subcore_axis_name="s",
                                  num_cores=NC, num_subcores=NS)

    @functools.partial(
        pl.kernel,
        out_type=jax.ShapeDtypeStruct((NC, LNUM, N, D), jnp.float32),
        mesh=mesh,
        scratch_types=[
            pltpu.VMEM((NCH_S, CH), jnp.int32),
            pltpu.VMEM((NCH_S, CH), jnp.int32),
            pltpu.VMEM((NCH_S, CH), jnp.float32),
            pltpu.VMEM((CH, D), jnp.float32),
            pltpu.VMEM((CH, D), jnp.float32),
            pltpu.VMEM((D,), jnp.float32),
            pltpu.VMEM_SHARED((NP, D), jnp.float32),
            pltpu.SemaphoreType.DMA,
            pltpu.SemaphoreType.DMA,
        ],
    )
    def body(tab_hbm, src_hbm, dst_hbm, ea_hbm, w_hbm, out_hbm,
             src_v, dst_v, ea_v, rows_v, rows2_v, w_v, agg_sh, gsem_a,
             gsem_b):
        c = lax.axis_index("c")
        s = lax.axis_index("s")
        wid = c * NS + s

        pltpu.sync_copy(w_hbm.at[0], w_v)
        w_vecs = [w_v[pl.ds(16 * j, 16)] for j in range(D // VLANES)]
        zv = jnp.zeros((VLANES,), jnp.float32)

        def _zero_rows(r, carry):
            for j in range(D // VLANES):
                rows_v[r, pl.ds(16 * j, 16)] = zv
            return carry

        def _layer(l, lcarry):
            lax.fori_loop(0, CH, _zero_rows, 0)
            for k in range(RPT // ZR):
                pltpu.sync_copy(
                    rows_v.at[pl.ds(0, ZR)],
                    agg_sh.at[pl.ds(s * RPT + k * ZR, ZR)])

            @pl.when(s == NS - 1)
            def _():
                pltpu.sync_copy(rows_v.at[pl.ds(0, N - NS * RPT)],
                                agg_sh.at[pl.ds(NS * RPT, N - NS * RPT)])
            plsc.subcore_barrier()

            def _compute(buf, g):
                def _edge16(i16, icarry):
                    ev = ea_v[g, pl.ds(i16 * VLANES, VLANES)]
                    ev = lax.min(lax.max(ev, 0.0), 1.0)
                    for k in range(VLANES):
                        e = ev[k]
                        ri = i16 * VLANES + k
                        for j in range(D // VLANES):
                            r = buf[ri, pl.ds(16 * j, 16)]
                            buf[ri, pl.ds(16 * j, 16)] = lax.max(
                                r + e * w_vecs[j], 0.0)
                    return icarry
                lax.fori_loop(0, CH // VLANES, _edge16, 0)

            def _sub(sb, scarry):
                pltpu.sync_copy(src_hbm.at[l, wid, sb], src_v)
                pltpu.sync_copy(dst_hbm.at[l, wid, sb], dst_v)
                pltpu.sync_copy(ea_hbm.at[l, wid, sb], ea_v)
                pltpu.async_copy(tab_hbm.at[src_v.at[0]], rows_v, gsem_a)

                def _pair(p, carry):
                    g0 = 2 * p
                    g1 = g0 + 1
                    pltpu.async_copy(
                        tab_hbm.at[src_v.at[g1]], rows2_v, gsem_b)
                    pltpu.make_async_copy(
                        tab_hbm.at[src_v.at[g0]], rows_v, gsem_a).wait()
                    _compute(rows_v, g0)
                    pltpu.sync_copy(rows_v, agg_sh.at[dst_v.at[g0]],
                                    add=True)
                    pltpu.async_copy(
                        tab_hbm.at[src_v.at[g0 + 2]], rows_v, gsem_a)
                    pltpu.make_async_copy(
                        tab_hbm.at[src_v.at[g1]], rows2_v, gsem_b).wait()
                    _compute(rows2_v, g1)
                    pltpu.sync_copy(rows2_v, agg_sh.at[dst_v.at[g1]],
                                    add=True)
                    return carry
                lax.fori_loop(0, NCH_S // 2, _pair, 0)
                pltpu.make_async_copy(
                    tab_hbm.at[src_v.at[NCH_S - 1]], rows_v, gsem_a).wait()
                _compute(rows_v, NCH_S - 1)
                pltpu.sync_copy(rows_v, agg_sh.at[dst_v.at[NCH_S - 1]],
                                add=True)
                return scarry
            lax.fori_loop(0, SSPLIT, _sub, 0)
            plsc.subcore_barrier()
            pltpu.sync_copy(agg_sh.at[pl.ds(s * RPT, RPT)],
                            out_hbm.at[c, l, pl.ds(s * RPT, RPT)])

            @pl.when(s == NS - 1)
            def _():
                pltpu.sync_copy(
                    agg_sh.at[pl.ds(NS * RPT, N - NS * RPT)],
                    out_hbm.at[c, l, pl.ds(NS * RPT, N - NS * RPT)])
            plsc.subcore_barrier()
            return lcarry
        lax.fori_loop(0, LNUM, _layer, 0)

    return body(tableb, src4, dst4, ea4, w_row)


NBM = 2000


def _prep_body(x_ref, b_ref, out_ref):
    out_ref[...] = x_ref[...] + b_ref[...][None]


def _prep_call(x, be1):
    return pl.pallas_call(
        _prep_body,
        grid=(LNUM, N // NBM),
        in_specs=[
            pl.BlockSpec((1, NBM, D), lambda l, i: (l, i, 0)),
            pl.BlockSpec((1, D), lambda l, i: (0, 0)),
        ],
        out_specs=pl.BlockSpec((1, NBM, D), lambda l, i: (l, i, 0)),
        out_shape=jax.ShapeDtypeStruct((LNUM, N, D), jnp.float32),
    )(x, be1)


def _mlp1_body(xb_ref, parts_ref, be1_ref, w1_ref, b1_ref, w2_ref, b2_ref,
               be2_ref, out_ref):
    h0 = (xb_ref[0] - be1_ref[...] + parts_ref[0, 0] + parts_ref[1, 0])
    a = jnp.maximum(
        jnp.dot(h0, w1_ref[...], preferred_element_type=jnp.float32)
        + b1_ref[...], 0.0)
    h1 = jnp.maximum(
        jnp.dot(a, w2_ref[...], preferred_element_type=jnp.float32)
        + b2_ref[...], 0.0)
    out_ref[0] = h1 + be2_ref[...]


def _mlp1_call(xb, parts, be1, w1t, b1, w2t, b2, be2):
    return pl.pallas_call(
        _mlp1_body,
        grid=(LNUM, N // NBM),
        in_specs=[
            pl.BlockSpec((1, NBM, D), lambda l, i: (l, i, 0)),
            pl.BlockSpec((2, 1, NBM, D), lambda l, i: (0, l, i, 0)),
            pl.BlockSpec((1, D), lambda l, i: (0, 0)),
            pl.BlockSpec((D, GH), lambda l, i: (0, 0)),
            pl.BlockSpec((1, GH), lambda l, i: (0, 0)),
            pl.BlockSpec((GH, GH), lambda l, i: (0, 0)),
            pl.BlockSpec((1, GH), lambda l, i: (0, 0)),
            pl.BlockSpec((1, GH), lambda l, i: (0, 0)),
        ],
        out_specs=pl.BlockSpec((1, NBM, GH), lambda l, i: (l, i, 0)),
        out_shape=jax.ShapeDtypeStruct((LNUM, N, GH), jnp.float32),
    )(xb, parts, be1, w1t, b1, w2t, b2, be2)


def _mlp2_body(h1b_ref, parts_ref, batch_ref, be2_ref, w1_ref, b1_ref,
               w2_ref, b2_ref, seg_ref, cnt_ref):
    l = pl.program_id(0)
    i = pl.program_id(1)
    h0 = (h1b_ref[0] - be2_ref[...] + parts_ref[0, 0] + parts_ref[1, 0])
    a = jnp.maximum(
        jnp.dot(h0, w1_ref[...], preferred_element_type=jnp.float32)
        + b1_ref[...], 0.0)
    h2 = (jnp.dot(a, w2_ref[...], preferred_element_type=jnp.float32)
          + b2_ref[...])
    b_blk = batch_ref[0, 0, :]
    io = lax.broadcasted_iota(jnp.int32, (NSEG, NBM), 0)
    oh = (b_blk[None, :] == io).astype(jnp.float32)
    sp = jnp.dot(oh, h2, preferred_element_type=jnp.float32)

    @pl.when(i == 0)
    def _():
        seg_ref[0] = sp

    @pl.when(i != 0)
    def _():
        seg_ref[0] = seg_ref[0] + sp

    cb = jnp.broadcast_to(jnp.sum(oh, axis=1, keepdims=True), (NSEG, GH))

    @pl.when((l == 0) & (i == 0))
    def _():
        cnt_ref[...] = cb

    @pl.when((l == 0) & (i != 0))
    def _():
        cnt_ref[...] = cnt_ref[...] + cb


def _mlp2_call(h1b, parts2, batch3, be2, w1t, b1, w2t, b2):
    return pl.pallas_call(
        _mlp2_body,
        grid=(LNUM, N // NBM),
        in_specs=[
            pl.BlockSpec((1, NBM, GH), lambda l, i: (l, i, 0)),
            pl.BlockSpec((2, 1, NBM, GH), lambda l, i: (0, l, i, 0)),
            pl.BlockSpec((1, 1, NBM), lambda l, i: (i, 0, 0)),
            pl.BlockSpec((1, GH), lambda l, i: (0, 0)),
            pl.BlockSpec((GH, GH), lambda l, i: (0, 0)),
            pl.BlockSpec((1, GH), lambda l, i: (0, 0)),
            pl.BlockSpec((GH, GH), lambda l, i: (0, 0)),
            pl.BlockSpec((1, GH), lambda l, i: (0, 0)),
        ],
        out_specs=[
            pl.BlockSpec((1, NSEG, GH), lambda l, i: (l, 0, 0)),
            pl.BlockSpec((NSEG, GH), lambda l, i: (0, 0)),
        ],
        out_shape=[
            jax.ShapeDtypeStruct((LNUM, NSEG, GH), jnp.float32),
            jax.ShapeDtypeStruct((NSEG, GH), jnp.float32),
        ],
    )(h1b, parts2, batch3, be2, w1t, b1, w2t, b2)


def _head_body(seg_ref, cnt_ref, mask_ref,
               wih1f_ref, whh1f_ref, bi1f_ref,
               wih1b_ref, whh1b_ref, bi1b_ref,
               wih2f_ref, whh2f_ref, bi2f_ref,
               wih2b_ref, whh2b_ref, bi2b_ref,
               fc1_ref, fb1_ref, fc2_ref, fb2_ref, w3_ref, b3_ref,
               out_ref):
    cnt = jnp.maximum(cnt_ref[...], 1.0)
    xs = [seg_ref[t] / cnt * mask_ref[t] for t in range(LNUM)]

    def lstm_dir(inp, wih_ref, whh_ref, bias_ref, reverse):
        h = jnp.zeros((NSEG, LH), jnp.float32)
        c = jnp.zeros((NSEG, LH), jnp.float32)
        ys = [None] * LNUM
        order = range(LNUM - 1, -1, -1) if reverse else range(LNUM)
        for t in order:
            g = (jnp.dot(inp[t], wih_ref[...],
                         preferred_element_type=jnp.float32)
                 + jnp.dot(h, whh_ref[...],
                           preferred_element_type=jnp.float32)
                 + bias_ref[...])
            ig = jax.nn.sigmoid(g[:, 0:LH])
            fg = jax.nn.sigmoid(g[:, LH:2 * LH])
            gg = jnp.tanh(g[:, 2 * LH:3 * LH])
            og = jax.nn.sigmoid(g[:, 3 * LH:4 * LH])
            c = fg * c + ig * gg
            h = og * jnp.tanh(c)
            ys[t] = h
        return ys, h

    ysf, _ = lstm_dir(xs, wih1f_ref, whh1f_ref, bi1f_ref, False)
    ysb, _ = lstm_dir(xs, wih1b_ref, whh1b_ref, bi1b_ref, True)
    xs2 = [jnp.concatenate([ysf[t], ysb[t]], axis=1) for t in range(LNUM)]
    _, hf2 = lstm_dir(xs2, wih2f_ref, whh2f_ref, bi2f_ref, False)
    _, hb2 = lstm_dir(xs2, wih2b_ref, whh2b_ref, bi2b_ref, True)
    final = jnp.concatenate([hf2, hb2], axis=1)
    z = jnp.maximum(
        jnp.dot(final, fc1_ref[...], preferred_element_type=jnp.float32)
        + fb1_ref[...], 0.0)
    z = jnp.maximum(
        jnp.dot(z, fc2_ref[...], preferred_element_type=jnp.float32)
        + fb2_ref[...], 0.0)
    res = jnp.sum(z * w3_ref[...], axis=1, keepdims=True)
    out_ref[...] = jnp.broadcast_to(res, (NSEG, GH)) + b3_ref[...]


def _head_call(seg, cnt, mask, lstm_w, fc1t, fb1, fc2t, fb2, w3, b3b):
    return pl.pallas_call(
        _head_body,
        out_shape=jax.ShapeDtypeStruct((NSEG, GH), jnp.float32),
    )(seg, cnt, mask, *lstm_w, fc1t, fb1, fc2t, fb2, w3, b3b)


def kernel(x, edge_index, edge_attr, batch, num_layers, params):
    f32 = jnp.float32
    pad = ((0, 0), (0, 0), (0, EPWP - EPW))
    src = jnp.pad(edge_index[:, 0, :].astype(jnp.int32).reshape(
        LNUM, NW, EPW), pad)
    dst = jnp.pad(edge_index[:, 1, :].astype(jnp.int32).reshape(
        LNUM, NW, EPW), pad, constant_values=N)
    ea = jnp.pad(edge_attr[:, :, 0].reshape(LNUM, NW, EPW), pad)
    offs = (jnp.arange(LNUM, dtype=jnp.int32) * N)[:, None, None]
    src4 = (src + offs).reshape(LNUM, NW, SSPLIT, NCH_S, CH)
    dst4 = dst.reshape(LNUM, NW, SSPLIT, NCH_S, CH)
    ea4 = ea.reshape(LNUM, NW, SSPLIT, NCH_S, CH)
    batch3 = batch.astype(jnp.int32).reshape(N // NBM, 1, NBM)

    we1, be1 = params['lin_edge1']
    we2, be2 = params['lin_edge2']
    we1r = we1.reshape(1, D)
    we2r = we2.reshape(1, GH)
    be1r = be1.reshape(1, D)
    be2r = be2.reshape(1, GH)
    (w11, b11), (w12, b12) = params['mlp1']
    (w21, b21), (w22, b22) = params['mlp2']
    w11t, w12t = w11.T, w12.T
    w21t, w22t = w21.T, w22.T
    b11r, b12r = b11.reshape(1, GH), b12.reshape(1, GH)
    b21r, b22r = b21.reshape(1, GH), b22.reshape(1, GH)

    lstm_w = []
    for (lp, inf) in ((params['lstm'][0][0], GH), (params['lstm'][0][1], GH),
                      (params['lstm'][1][0], 2 * LH),
                      (params['lstm'][1][1], 2 * LH)):
        Wih, Whh, bih, bhh = lp
        lstm_w += [Wih.T, Whh.T, (bih + bhh).reshape(1, 4 * LH)]
    fc1, fb1 = params['fc1']
    fc2, fb2 = params['fc2']
    fc3, fb3 = params['fc3']
    fc1t, fc2t = fc1.T, fc2.T
    fb1r, fb2r = fb1.reshape(1, 128), fb2.reshape(1, 64)
    w3r = fc3.reshape(1, 64)
    b3b = jnp.broadcast_to(fb3.reshape(1, 1), (1, GH))

    mask = (jnp.arange(LNUM) < num_layers).astype(f32)
    mask2 = jnp.broadcast_to(mask[:, None], (LNUM, GH))

    xb = _prep_call(x, be1r)
    parts1 = _sc_conv_call(xb.reshape(LNUM * N, D), src4, dst4, ea4, we1r)
    h1b = _mlp1_call(xb, parts1, be1r, w11t, b11r, w12t, b12r, be2r)
    parts2 = _sc_conv_call(h1b.reshape(LNUM * N, GH), src4, dst4, ea4, we2r)
    seg, cnt = _mlp2_call(h1b, parts2, batch3, be2r,
                          w21t, b21r, w22t, b22r)
    out = _head_call(seg, cnt, mask2, lstm_w, fc1t, fb1r, fc2t, fb2r,
                     w3r, b3b)
    return out[:, :1]
